# baseline (speedup 1.0000x reference)
import functools

import numpy as np

import jax
import jax.numpy as jnp
from jax import lax
from jax.experimental import pallas as pl
from jax.experimental.pallas import tpu as pltpu
from jax.experimental.pallas import tpu_sc as plsc

N = 10000
E = 320000
F = 128
G = 64

NC = 2
NS = 16
NW = NC * NS
L = 16

EPW = E // NW
C = 80
NCH = EPW // C
RB = 80
NRB = N // RB
ZROUNDS = (NRB + NS - 1) // NS

_MESH = plsc.VectorSubcoreMesh(core_axis_name="c", subcore_axis_name="s")


@functools.partial(
    pl.kernel,
    out_type=jax.ShapeDtypeStruct((NW, N), jnp.float32),
    mesh=_MESH,
    scratch_types=[
        pltpu.VMEM((EPW,), jnp.int32),
        pltpu.VMEM((EPW,), jnp.float32),
        pltpu.VMEM((N,), jnp.float32),
    ],
    compiler_params=pltpu.CompilerParams(needs_layout_passes=False),
)
def _sc_deg(col_h, ew_h, out_h, colv, ewv, degv):
    cid = lax.axis_index("c")
    sid = lax.axis_index("s")
    wid = sid * NC + cid
    base = wid * EPW

    def zero(i, carry):
        degv[pl.ds(i * L, L)] = jnp.zeros((L,), jnp.float32)
        return carry

    lax.fori_loop(0, N // L, zero, 0)

    pltpu.sync_copy(col_h.at[pl.ds(base, EPW)], colv)
    pltpu.sync_copy(ew_h.at[pl.ds(base, EPW)], ewv)

    def body(i, carry):
        idx = colv[pl.ds(i * L, L)]
        w = ewv[pl.ds(i * L, L)]
        plsc.addupdate_scatter(degv, [idx], w)
        return carry

    lax.fori_loop(0, EPW // L, body, 0)
    pltpu.sync_copy(degv, out_h.at[wid])


@functools.partial(
    pl.kernel,
    out_type=jax.ShapeDtypeStruct((NC, N, F), jnp.float32),
    mesh=_MESH,
    scratch_types=[
        pltpu.VMEM((EPW,), jnp.int32),
        pltpu.VMEM((C,), jnp.float32),
        pltpu.VMEM((C,), jnp.float32),
        pltpu.VMEM((C,), jnp.int32),
        pltpu.VMEM((C,), jnp.int32),
        pltpu.VMEM((C, F), jnp.bfloat16),
        pltpu.VMEM((C, F), jnp.bfloat16),
        pltpu.VMEM((C, F), jnp.float32),
        pltpu.VMEM((C, F), jnp.float32),
        pltpu.SemaphoreType.DMA,
        pltpu.SemaphoreType.DMA,
        pltpu.SemaphoreType.DMA,
        pltpu.SemaphoreType.DMA,
        pltpu.VMEM_SHARED((N, F), jnp.float32),
    ],
    compiler_params=pltpu.CompilerParams(
        needs_layout_passes=False, use_tc_tiling_on_sc=False),
)
def _sc_agg(y_h, row_h, col_h, ew_h, agg_h, rowv, ewb0, ewb1, colb0, colb1,
            gbuf0, gbuf1, mbuf0, mbuf1, sem_g, sem_s, sem_c, sem_e,
            shared):
    cid = lax.axis_index("c")
    sid = lax.axis_index("s")
    wid = sid * NC + cid
    base = wid * EPW

    pltpu.sync_copy(row_h.at[pl.ds(base, EPW)], rowv)
    pltpu.async_copy(col_h.at[pl.ds(base, C)], colb0, sem_c)
    pltpu.async_copy(ew_h.at[pl.ds(base, C)], ewb0, sem_e)
    pltpu.async_copy(y_h.at[rowv.at[pl.ds(0, C)]], gbuf0, sem_g)

    def zzero(i, carry):
        for j in range(F // L):
            mbuf1[i, pl.ds(j * L, L)] = jnp.zeros((L,), jnp.float32)
        return carry

    lax.fori_loop(0, RB, zzero, 0)

    def szero(t, carry):
        k = sid + NS * t

        @pl.when(k < NRB)
        def _():
            pltpu.sync_copy(mbuf1, shared.at[pl.ds(k * RB, RB)])

        return carry

    lax.fori_loop(0, ZROUNDS, szero, 0)
    plsc.subcore_barrier()

    gb = (gbuf0, gbuf1)
    mb = (mbuf0, mbuf1)
    cb = (colb0, colb1)
    eb = (ewb0, ewb1)
    himask = jnp.full((L,), -65536, dtype=jnp.int32)

    def pair(t, carry):
        for b in (0, 1):
            k = 2 * t + b
            cur = gb[b]
            nxt = gb[1 - b]
            curm = mb[b]
            curc = cb[b]
            nxtc = cb[1 - b]
            cure = eb[b]
            nxte = eb[1 - b]

            @pl.when(k < NCH)
            def _():
                pltpu.make_async_copy(
                    y_h.at[rowv.at[pl.ds(k * C, C)]], cur, sem_g).wait()
                pltpu.make_async_copy(
                    col_h.at[pl.ds(base + k * C, C)], curc, sem_c).wait()
                pltpu.make_async_copy(
                    ew_h.at[pl.ds(base + k * C, C)], cure, sem_e).wait()

                @pl.when(k + 1 < NCH)
                def _():
                    pltpu.async_copy(
                        y_h.at[rowv.at[pl.ds((k + 1) * C, C)]], nxt, sem_g)

                @pl.when(k >= 1)
                def _():
                    pltpu.make_async_copy(curm, shared.at[curc], sem_s).wait()

                @pl.when(k + 1 < NCH)
                def _():
                    pltpu.async_copy(
                        col_h.at[pl.ds(base + (k + 1) * C, C)], nxtc, sem_c)
                    pltpu.async_copy(
                        ew_h.at[pl.ds(base + (k + 1) * C, C)], nxte, sem_e)

                def group(g, icarry):
                    wv = cure[pl.ds(g * L, L)]
                    for lane in range(L):
                        i = g * L + lane
                        w = wv[lane]
                        for j in range(F // (2 * L)):
                            v = plsc.bitcast(
                                cur[i, pl.ds(j * 2 * L, 2 * L)], jnp.int32)
                            lo = plsc.bitcast(v << 16, jnp.float32)
                            hi = plsc.bitcast(v & himask, jnp.float32)
                            curm[i, pl.ds(j * 2 * L, L)] = lo * w
                            curm[i, pl.ds(j * 2 * L + L, L)] = hi * w
                    return icarry

                lax.fori_loop(0, C // L, group, 0)
                pltpu.async_copy(curm, shared.at[curc], sem_s, add=True)

        return carry

    lax.fori_loop(0, (NCH + 1) // 2, pair, 0)
    pltpu.make_async_copy(mbuf0, shared.at[colb0], sem_s).wait()
    plsc.subcore_barrier()

    def wout(t, carry):
        k = sid + NS * t

        @pl.when(k < NRB)
        def _():
            pltpu.sync_copy(shared.at[pl.ds(k * RB, RB)],
                            agg_h.at[cid, pl.ds(k * RB, RB)])

        return carry

    lax.fori_loop(0, ZROUNDS, wout, 0)


R = 1000
NB = N // R


def _pre_body(degp_ref, x_ref, w_ref, p_ref, y_ref, yb_ref, dinv_ref):
    deg = jnp.sum(degp_ref[...], axis=1, keepdims=True) + 1.0
    dinv = jnp.where(deg > 0, lax.rsqrt(deg), 0.0)
    xw = jnp.dot(x_ref[...], w_ref[...], preferred_element_type=jnp.float32)
    y = dinv * xw
    y_ref[...] = y
    yb_ref[...] = jnp.dot(
        y, p_ref[...], preferred_element_type=jnp.float32
    ).astype(jnp.bfloat16)
    dinv_ref[...] = dinv


def _mid_body(aggp_ref, y_ref, dinv_ref, b_ref, w_ref, p_ref, y2_ref,
              y2b_ref):
    dinv = dinv_ref[...]
    agg = aggp_ref[0] + aggp_ref[1]
    h = jnp.maximum(dinv * (agg + y_ref[...]) + b_ref[...], 0.0)
    xw = jnp.dot(h, w_ref[...], preferred_element_type=jnp.float32)
    y2 = dinv * xw
    y2_ref[...] = y2
    y2b_ref[...] = jnp.dot(
        y2, p_ref[...], preferred_element_type=jnp.float32
    ).astype(jnp.bfloat16)


def _post_body(aggp_ref, y_ref, dinv_ref, b_ref, batch_ref, out_ref):
    dinv = dinv_ref[...]
    agg = aggp_ref[0] + aggp_ref[1]
    h2 = dinv * (agg + y_ref[...]) + b_ref[...]
    gids = lax.broadcasted_iota(jnp.int32, (R, G), 1)
    mask = (batch_ref[...] == gids).astype(jnp.float32)

    @pl.when(pl.program_id(0) == 0)
    def _():
        out_ref[...] = jnp.zeros_like(out_ref)

    out_ref[...] += lax.dot_general(
        mask, h2, (((0,), (0,)), ((), ())),
        preferred_element_type=jnp.float32)


_pre = pl.pallas_call(
    _pre_body,
    grid=(NB,),
    in_specs=[
        pl.BlockSpec((R, NW), lambda i: (i, 0)),
        pl.BlockSpec((R, F), lambda i: (i, 0)),
        pl.BlockSpec((F, F), lambda i: (0, 0)),
        pl.BlockSpec((F, F), lambda i: (0, 0)),
    ],
    out_specs=[
        pl.BlockSpec((R, F), lambda i: (i, 0)),
        pl.BlockSpec((R, F), lambda i: (i, 0)),
        pl.BlockSpec((R, 1), lambda i: (i, 0)),
    ],
    out_shape=[
        jax.ShapeDtypeStruct((N, F), jnp.float32),
        jax.ShapeDtypeStruct((N, F), jnp.bfloat16),
        jax.ShapeDtypeStruct((N, 1), jnp.float32),
    ],
)

_mid = pl.pallas_call(
    _mid_body,
    grid=(NB,),
    in_specs=[
        pl.BlockSpec((NC, R, F), lambda i: (0, i, 0)),
        pl.BlockSpec((R, F), lambda i: (i, 0)),
        pl.BlockSpec((R, 1), lambda i: (i, 0)),
        pl.BlockSpec((1, F), lambda i: (0, 0)),
        pl.BlockSpec((F, F), lambda i: (0, 0)),
        pl.BlockSpec((F, F), lambda i: (0, 0)),
    ],
    out_specs=[
        pl.BlockSpec((R, F), lambda i: (i, 0)),
        pl.BlockSpec((R, F), lambda i: (i, 0)),
    ],
    out_shape=[
        jax.ShapeDtypeStruct((N, F), jnp.float32),
        jax.ShapeDtypeStruct((N, F), jnp.bfloat16),
    ],
)

_post = pl.pallas_call(
    _post_body,
    grid=(NB,),
    in_specs=[
        pl.BlockSpec((NC, R, F), lambda i: (0, i, 0)),
        pl.BlockSpec((R, F), lambda i: (i, 0)),
        pl.BlockSpec((R, 1), lambda i: (i, 0)),
        pl.BlockSpec((1, F), lambda i: (0, 0)),
        pl.BlockSpec((R, 1), lambda i: (i, 0)),
    ],
    out_specs=pl.BlockSpec((G, F), lambda i: (0, 0)),
    out_shape=jax.ShapeDtypeStruct((G, F), jnp.float32),
)


def _perm_matrix():
    p = np.zeros((F, F), dtype=np.float32)
    for blk in range(F // 32):
        for j in range(16):
            p[32 * blk + j, 32 * blk + 2 * j] = 1.0
            p[32 * blk + 16 + j, 32 * blk + 2 * j + 1] = 1.0
    return jnp.asarray(p)


def kernel(x, edge_index, edge_weight, batch, W1, b1, W2, b2):
    row = edge_index[0]
    col = edge_index[1]
    perm = _perm_matrix()
    deg_parts = _sc_deg(col, edge_weight)
    degp = deg_parts.T
    y1, y1b, dinv = _pre(degp, x, W1, perm)
    agg1 = _sc_agg(y1b, row, col, edge_weight)
    y2, y2b = _mid(agg1, y1, dinv, b1.reshape(1, F), W2, perm)
    agg2 = _sc_agg(y2b, row, col, edge_weight)
    out = _post(agg2, y2, dinv, b2.reshape(1, F), batch.reshape(N, 1))
    return out


# speedup vs baseline: 1.7908x; 1.7908x over previous
import functools

import jax
import jax.numpy as jnp
from jax import lax
from jax.experimental import pallas as pl
from jax.experimental.pallas import tpu as pltpu
from jax.experimental.pallas import tpu_sc as plsc

N = 10000
E = 320000
F = 128
G = 64

NC = 2
NS = 16
NW = NC * NS
L = 16

EPW = E // NW
C = 80
NCH = EPW // C
RB = 80
NRB = N // RB
ZROUNDS = (NRB + NS - 1) // NS

_MESH = plsc.VectorSubcoreMesh(core_axis_name="c", subcore_axis_name="s")


@functools.partial(
    pl.kernel,
    out_type=jax.ShapeDtypeStruct((NW, N), jnp.float32),
    mesh=_MESH,
    scratch_types=[
        pltpu.VMEM((EPW,), jnp.int32),
        pltpu.VMEM((EPW,), jnp.float32),
        pltpu.VMEM((N,), jnp.float32),
    ],
    compiler_params=pltpu.CompilerParams(needs_layout_passes=False),
)
def _sc_deg(col_h, ew_h, out_h, colv, ewv, degv):
    cid = lax.axis_index("c")
    sid = lax.axis_index("s")
    wid = sid * NC + cid
    base = wid * EPW

    def zero(i, carry):
        degv[pl.ds(i * L, L)] = jnp.zeros((L,), jnp.float32)
        return carry

    lax.fori_loop(0, N // L, zero, 0)

    pltpu.sync_copy(col_h.at[pl.ds(base, EPW)], colv)
    pltpu.sync_copy(ew_h.at[pl.ds(base, EPW)], ewv)

    def body(i, carry):
        idx = colv[pl.ds(i * L, L)]
        w = ewv[pl.ds(i * L, L)]
        plsc.addupdate_scatter(degv, [idx], w)
        return carry

    lax.fori_loop(0, EPW // L, body, 0)
    pltpu.sync_copy(degv, out_h.at[wid])


@functools.partial(
    pl.kernel,
    out_type=jax.ShapeDtypeStruct((NC, N, F), jnp.float32),
    mesh=_MESH,
    scratch_types=[
        pltpu.VMEM((EPW,), jnp.int32),
        pltpu.VMEM((EPW,), jnp.float32),
        pltpu.VMEM((C,), jnp.int32),
        pltpu.VMEM((C,), jnp.int32),
        pltpu.VMEM((C, F), jnp.float32),
        pltpu.VMEM((C, F), jnp.float32),
        pltpu.VMEM_SHARED((N, F), jnp.float32),
        pltpu.SemaphoreType.DMA,
        pltpu.SemaphoreType.DMA,
        pltpu.SemaphoreType.DMA,
    ],
    compiler_params=pltpu.CompilerParams(needs_layout_passes=False),
)
def _sc_agg(y_h, row_h, col_h, ew_h, agg_h, rowv, ewv, colb0, colb1,
            gbuf0, gbuf1, shared, sem_g, sem_s, sem_c):
    cid = lax.axis_index("c")
    sid = lax.axis_index("s")
    wid = sid * NC + cid
    base = wid * EPW

    pltpu.sync_copy(row_h.at[pl.ds(base, EPW)], rowv)
    pltpu.sync_copy(ew_h.at[pl.ds(base, EPW)], ewv)
    pltpu.async_copy(col_h.at[pl.ds(base, C)], colb0, sem_c)
    pltpu.async_copy(y_h.at[rowv.at[pl.ds(0, C)]], gbuf0, sem_g)

    def zzero(i, carry):
        for j in range(F // L):
            gbuf1[i, pl.ds(j * L, L)] = jnp.zeros((L,), jnp.float32)
        return carry

    lax.fori_loop(0, RB, zzero, 0)

    def szero(t, carry):
        k = sid + NS * t

        @pl.when(k < NRB)
        def _():
            pltpu.sync_copy(gbuf1, shared.at[pl.ds(k * RB, RB)])

        return carry

    lax.fori_loop(0, ZROUNDS, szero, 0)
    plsc.subcore_barrier()

    gb = (gbuf0, gbuf1)
    cb = (colb0, colb1)

    def pair(t, carry):
        for b in (0, 1):
            k = 2 * t + b
            cur = gb[b]
            nxt = gb[1 - b]
            curc = cb[b]
            nxtc = cb[1 - b]

            @pl.when(k < NCH)
            def _():
                pltpu.make_async_copy(
                    y_h.at[rowv.at[pl.ds(k * C, C)]], cur, sem_g).wait()
                pltpu.make_async_copy(
                    col_h.at[pl.ds(base + k * C, C)], curc, sem_c).wait()

                @pl.when(k + 1 < NCH)
                def _():
                    @pl.when(k >= 1)
                    def _():
                        pltpu.make_async_copy(
                            nxt, shared.at[curc], sem_s).wait()

                    pltpu.async_copy(
                        y_h.at[rowv.at[pl.ds((k + 1) * C, C)]], nxt, sem_g)
                    pltpu.async_copy(
                        col_h.at[pl.ds(base + (k + 1) * C, C)], nxtc, sem_c)

                def group(g, icarry):
                    wv = ewv[pl.ds(k * C + g * L, L)]
                    for lane in range(L):
                        i = g * L + lane
                        w = wv[lane]
                        for j in range(F // L):
                            cur[i, pl.ds(j * L, L)] = (
                                cur[i, pl.ds(j * L, L)] * w)
                    return icarry

                lax.fori_loop(0, C // L, group, 0)
                pltpu.async_copy(cur, shared.at[curc], sem_s, add=True)

        return carry

    lax.fori_loop(0, (NCH + 1) // 2, pair, 0)
    pltpu.make_async_copy(gbuf0, shared.at[colb0], sem_s).wait()
    pltpu.make_async_copy(gbuf1, shared.at[colb0], sem_s).wait()
    plsc.subcore_barrier()

    def wout(t, carry):
        k = sid + NS * t

        @pl.when(k < NRB)
        def _():
            pltpu.sync_copy(shared.at[pl.ds(k * RB, RB)],
                            agg_h.at[cid, pl.ds(k * RB, RB)])

        return carry

    lax.fori_loop(0, ZROUNDS, wout, 0)


R = 1000
NB = N // R


def _pre_body(degp_ref, x_ref, w_ref, y_ref, dinv_ref):
    deg = jnp.sum(degp_ref[...], axis=1, keepdims=True) + 1.0
    dinv = jnp.where(deg > 0, lax.rsqrt(deg), 0.0)
    xw = jnp.dot(x_ref[...], w_ref[...], preferred_element_type=jnp.float32)
    y_ref[...] = dinv * xw
    dinv_ref[...] = dinv


def _mid_body(aggp_ref, y_ref, dinv_ref, b_ref, w_ref, y2_ref):
    dinv = dinv_ref[...]
    agg = aggp_ref[0] + aggp_ref[1]
    h = jnp.maximum(dinv * (agg + y_ref[...]) + b_ref[...], 0.0)
    xw = jnp.dot(h, w_ref[...], preferred_element_type=jnp.float32)
    y2_ref[...] = dinv * xw


def _post_body(aggp_ref, y_ref, dinv_ref, b_ref, batch_ref, out_ref):
    dinv = dinv_ref[...]
    agg = aggp_ref[0] + aggp_ref[1]
    h2 = dinv * (agg + y_ref[...]) + b_ref[...]
    gids = lax.broadcasted_iota(jnp.int32, (R, G), 1)
    mask = (batch_ref[...] == gids).astype(jnp.float32)

    @pl.when(pl.program_id(0) == 0)
    def _():
        out_ref[...] = jnp.zeros_like(out_ref)

    out_ref[...] += lax.dot_general(
        mask, h2, (((0,), (0,)), ((), ())),
        preferred_element_type=jnp.float32)


_pre = pl.pallas_call(
    _pre_body,
    grid=(NB,),
    in_specs=[
        pl.BlockSpec((R, NW), lambda i: (i, 0)),
        pl.BlockSpec((R, F), lambda i: (i, 0)),
        pl.BlockSpec((F, F), lambda i: (0, 0)),
    ],
    out_specs=[
        pl.BlockSpec((R, F), lambda i: (i, 0)),
        pl.BlockSpec((R, 1), lambda i: (i, 0)),
    ],
    out_shape=[
        jax.ShapeDtypeStruct((N, F), jnp.float32),
        jax.ShapeDtypeStruct((N, 1), jnp.float32),
    ],
)

_mid = pl.pallas_call(
    _mid_body,
    grid=(NB,),
    in_specs=[
        pl.BlockSpec((NC, R, F), lambda i: (0, i, 0)),
        pl.BlockSpec((R, F), lambda i: (i, 0)),
        pl.BlockSpec((R, 1), lambda i: (i, 0)),
        pl.BlockSpec((1, F), lambda i: (0, 0)),
        pl.BlockSpec((F, F), lambda i: (0, 0)),
    ],
    out_specs=pl.BlockSpec((R, F), lambda i: (i, 0)),
    out_shape=jax.ShapeDtypeStruct((N, F), jnp.float32),
)

_post = pl.pallas_call(
    _post_body,
    grid=(NB,),
    in_specs=[
        pl.BlockSpec((NC, R, F), lambda i: (0, i, 0)),
        pl.BlockSpec((R, F), lambda i: (i, 0)),
        pl.BlockSpec((R, 1), lambda i: (i, 0)),
        pl.BlockSpec((1, F), lambda i: (0, 0)),
        pl.BlockSpec((R, 1), lambda i: (i, 0)),
    ],
    out_specs=pl.BlockSpec((G, F), lambda i: (0, 0)),
    out_shape=jax.ShapeDtypeStruct((G, F), jnp.float32),
)


def kernel(x, edge_index, edge_weight, batch, W1, b1, W2, b2):
    row = edge_index[0]
    col = edge_index[1]
    deg_parts = _sc_deg(col, edge_weight)
    degp = deg_parts.T
    y1, dinv = _pre(degp, x, W1)
    agg1 = _sc_agg(y1, row, col, edge_weight)
    y2 = _mid(agg1, y1, dinv, b1.reshape(1, F), W2)
    agg2 = _sc_agg(y2, row, col, edge_weight)
    out = _post(agg2, y2, dinv, b2.reshape(1, F), batch.reshape(N, 1))
    return out
